# Initial kernel scaffold; baseline (speedup 1.0000x reference)
import jax
import jax.numpy as jnp
from jax.experimental import pallas as pl


def kernel(boxes1, boxes2):
    raise NotImplementedError("write your pallas kernel here")


# speedup vs baseline: 1.5858x; 1.5858x over previous
import functools

import jax
import jax.numpy as jnp
from jax.experimental import pallas as pl

_N = 20000
_B = 512
_ROWS = 512


def _iou_kernel(b1_ref, b2t_ref, ious_ref, amax_ref):
    b1 = b1_ref[...]
    b2 = b2t_ref[...]

    im_a = b1[:, 0:1]
    x1a = b1[:, 1:2]
    y1a = b1[:, 2:3]
    x2a = b1[:, 3:4]
    y2a = b1[:, 4:5]

    im_b = b2[0:1, :]
    x1b = b2[1:2, :]
    y1b = b2[2:3, :]
    x2b = b2[3:4, :]
    y2b = b2[4:5, :]

    iw = jnp.maximum(jnp.minimum(x2a, x2b) - jnp.maximum(x1a, x1b) + 1.0, 0.0)
    ih = jnp.maximum(jnp.minimum(y2a, y2b) - jnp.maximum(y1a, y1b) + 1.0, 0.0)
    inter = iw * ih
    area_a = (x2a - x1a + 1.0) * (y2a - y1a + 1.0)
    area_b = (x2b - x1b + 1.0) * (y2b - y1b + 1.0)
    iou = inter / (area_a + area_b - inter)
    iou = jnp.where(im_a != im_b, 0.0, iou)
    ious_ref[...] = iou

    mx = jnp.max(iou, axis=1, keepdims=True)
    col = jax.lax.broadcasted_iota(jnp.int32, iou.shape, 1)
    amax_ref[...] = jnp.min(
        jnp.where(iou == mx, col, _B), axis=1, keepdims=True
    )


@functools.partial(jax.jit, static_argnames=())
def kernel(boxes1, boxes2):
    b2t = boxes2.T
    grid = (pl.cdiv(_N, _ROWS),)
    ious, amax = pl.pallas_call(
        _iou_kernel,
        grid=grid,
        in_specs=[
            pl.BlockSpec((_ROWS, 5), lambda i: (i, 0)),
            pl.BlockSpec((5, _B), lambda i: (0, 0)),
        ],
        out_specs=[
            pl.BlockSpec((_ROWS, _B), lambda i: (i, 0)),
            pl.BlockSpec((_ROWS, 1), lambda i: (i, 0)),
        ],
        out_shape=[
            jax.ShapeDtypeStruct((_N, _B), jnp.float32),
            jax.ShapeDtypeStruct((_N, 1), jnp.int32),
        ],
    )(boxes1, b2t)
    return amax.reshape(_N), ious


# speedup vs baseline: 1.7147x; 1.0813x over previous
import functools

import jax
import jax.numpy as jnp
from jax.experimental import pallas as pl
from jax.experimental.pallas import tpu as pltpu

_N = 20000
_B = 512
_ROWS = 2048


def _iou_kernel(b1_ref, b2t_ref, ious_ref, amax_ref):
    b1 = b1_ref[...]
    b2 = b2t_ref[...]

    im_a = b1[:, 0:1]
    x1a = b1[:, 1:2]
    y1a = b1[:, 2:3]
    x2a = b1[:, 3:4]
    y2a = b1[:, 4:5]

    im_b = b2[0:1, :]
    x1b = b2[1:2, :]
    y1b = b2[2:3, :]
    x2b = b2[3:4, :]
    y2b = b2[4:5, :]

    iw = jnp.maximum(jnp.minimum(x2a, x2b) - jnp.maximum(x1a, x1b) + 1.0, 0.0)
    ih = jnp.maximum(jnp.minimum(y2a, y2b) - jnp.maximum(y1a, y1b) + 1.0, 0.0)
    inter = iw * ih
    area_a = (x2a - x1a + 1.0) * (y2a - y1a + 1.0)
    area_b = (x2b - x1b + 1.0) * (y2b - y1b + 1.0)
    iou = inter / (area_a + area_b - inter)
    iou = jnp.where(im_a != im_b, 0.0, iou)
    ious_ref[...] = iou

    mx = jnp.max(iou, axis=1, keepdims=True)
    col = jax.lax.broadcasted_iota(jnp.int32, iou.shape, 1)
    amax_ref[...] = jnp.min(
        jnp.where(iou == mx, col, _B), axis=1, keepdims=True
    )


@functools.partial(jax.jit, static_argnames=())
def kernel(boxes1, boxes2):
    b2t = boxes2.T
    grid = (pl.cdiv(_N, _ROWS),)
    ious, amax = pl.pallas_call(
        _iou_kernel,
        grid=grid,
        in_specs=[
            pl.BlockSpec((_ROWS, 5), lambda i: (i, 0)),
            pl.BlockSpec((5, _B), lambda i: (0, 0)),
        ],
        out_specs=[
            pl.BlockSpec((_ROWS, _B), lambda i: (i, 0)),
            pl.BlockSpec((_ROWS, 1), lambda i: (i, 0)),
        ],
        out_shape=[
            jax.ShapeDtypeStruct((_N, _B), jnp.float32),
            jax.ShapeDtypeStruct((_N, 1), jnp.int32),
        ],
        compiler_params=pltpu.CompilerParams(
            dimension_semantics=("parallel",),
        ),
    )(boxes1, b2t)
    return amax.reshape(_N), ious
